# baseline (speedup 1.0000x reference)
import functools

import jax
import jax.numpy as jnp
from jax import lax
from jax.experimental import pallas as pl
from jax.experimental.pallas import tpu as pltpu
from jax.experimental.pallas import tpu_sc as plsc

N = 50000
NLEV = 4
DIMS = (16, 32, 64, 128)
DTOT = 240
NC, NS = 2, 16
NW = NC * NS
NA = 16
NB1, NB2 = 11, 5

ABLK = 64
NBLK_A = -(-N // ABLK)
LAST_A = N - ABLK
SLOTS_A = 2 * (-(-NBLK_A // (2 * NA)))

BBLK = 128
NBLK_B = -(-N // BBLK)
LAST_B = N - BBLK
SLOTS_B1 = 2 * (-(-NBLK_B // (2 * NB1)))
SLOTS_B2 = 2 * (-(-NBLK_B // (2 * NB2)))


def _body(clvt, w0t, w1t, w2t, w3, outt,
          ia0, ia1, ib0, ib1, w3ra, w3rb, oa0, oa1, ob0, ob1,
          st0, st1, st2, isem0, isem1, gsem0, gsem1, wsem0, wsem1):
    wid = lax.axis_index("c") * NS + lax.axis_index("s")
    iota = lax.iota(jnp.int32, 16)

    def dec(ref, n):
        for j in range(n // 16):
            ref[pl.ds(j * 16, 16)] = ref[pl.ds(j * 16, 16)] - 1


    def a_path():
        aid = wid

        def base_a(t):
            return lax.min(
                lax.min(lax.min(t, SLOTS_A - 1) * NA + aid, NBLK_A - 1)
                * ABLK, LAST_A)

        def transpose(wref, oref):
            def cg_body(cg, carry):
                cvec = iota + cg * 16
                for f in range(128):
                    fvec = jnp.full((16,), f, jnp.int32)
                    v = plsc.load_gather(wref, [cvec, fvec])
                    oref[f, pl.ds(cg * 16, 16)] = v
                return carry

            lax.fori_loop(0, ABLK // 16, cg_body, 0)

        def blk_body(t, carry):
            base = base_a(t)
            pltpu.sync_copy(clvt.at[pl.ds(3 * N + base, ABLK)], ia0)
            dec(ia0, ABLK)
            pltpu.async_copy(w3.at[ia0], w3ra, gsem0).wait()
            transpose(w3ra, oa0)
            pltpu.sync_copy(
                oa0, outt.at[pl.ds(112, 128), pl.ds(base, ABLK)])
            return carry

        lax.fori_loop(0, SLOTS_A, blk_body, 0)


    def b_path(bid, n_g, slots, levels, out_off, out_rows, w2_half):
        pltpu.sync_copy(w0t, st0)
        pltpu.sync_copy(w1t, st1)
        pltpu.sync_copy(w2t.at[pl.ds(w2_half * 64000, 64000)], st2)
        nlv = len(levels)
        ia = [(ib0.at[pl.ds(l * BBLK, BBLK)]) for l in range(nlv)]

        def base_b(t):
            return lax.min(
                lax.min(lax.min(t, slots - 1) * n_g + bid, NBLK_B - 1)
                * BBLK, LAST_B)

        def compute(irefs, oref):
            def cg_body(cg, carry):
                row = 0
                for (l, stage, nf, nv), iref in zip(levels, irefs):
                    ivec = iref[pl.ds(cg * 16, 16)]
                    for f in range(nf):
                        v = plsc.load_gather(stage, [ivec + (f * nv)])
                        oref[row + f, pl.ds(cg * 16, 16)] = v
                    row += nf
                return carry

            lax.fori_loop(0, BBLK // 16, cg_body, 0)

        def blk_body(t, carry):
            base = base_b(t)
            for (l, _, _, _), iref in zip(levels, ia):
                pltpu.async_copy(
                    clvt.at[pl.ds(l * N + base, BBLK)], iref, isem0)
            for (l, _, _, _), iref in zip(levels, ia):
                pltpu.make_async_copy(
                    clvt.at[pl.ds(l * N + base, BBLK)], iref, isem0).wait()
            for iref in ia:
                dec(iref, BBLK)
            compute(ia, ob0)
            pltpu.sync_copy(
                ob0.at[pl.ds(0, out_rows), :],
                outt.at[pl.ds(out_off, out_rows), pl.ds(base, BBLK)])
            return carry

        lax.fori_loop(0, slots, blk_body, 0)

    pl.when(wid < NA)(a_path)
    pl.when((wid >= NA) & (wid < NA + NB1))(
        lambda: b_path(wid - NA, NB1, SLOTS_B1,
                       ((0, st0, 16, 20), (1, st1, 32, 200),
                        (2, st2, 32, 2000)),
                       0, 80, 0))
    pl.when(wid >= NA + NB1)(
        lambda: b_path(wid - NA - NB1, NB2, SLOTS_B2,
                       ((2, st2, 32, 2000),),
                       80, 32, 1))


@jax.jit
def kernel(code_levels, W0, W1, W2, W3):
    mesh = plsc.VectorSubcoreMesh(core_axis_name="c", subcore_axis_name="s")
    f = pl.kernel(
        _body,
        out_type=jax.ShapeDtypeStruct((DTOT, N), jnp.float32),
        mesh=mesh,
        scratch_types=[
            pltpu.VMEM((ABLK,), jnp.int32),
            pltpu.VMEM((ABLK,), jnp.int32),
            pltpu.VMEM((3 * BBLK,), jnp.int32),
            pltpu.VMEM((3 * BBLK,), jnp.int32),
            pltpu.VMEM((ABLK, DIMS[3]), jnp.float32),
            pltpu.VMEM((ABLK, DIMS[3]), jnp.float32),
            pltpu.VMEM((128, ABLK), jnp.float32),
            pltpu.VMEM((128, ABLK), jnp.float32),
            pltpu.VMEM((80, BBLK), jnp.float32),
            pltpu.VMEM((80, BBLK), jnp.float32),
            pltpu.VMEM((DIMS[0] * 20,), jnp.float32),
            pltpu.VMEM((DIMS[1] * 200,), jnp.float32),
            pltpu.VMEM((32 * 2000,), jnp.float32),
            pltpu.SemaphoreType.DMA,
            pltpu.SemaphoreType.DMA,
            pltpu.SemaphoreType.DMA,
            pltpu.SemaphoreType.DMA,
            pltpu.SemaphoreType.DMA,
            pltpu.SemaphoreType.DMA,
        ],
        compiler_params=pltpu.CompilerParams(
            use_tc_tiling_on_sc=False, needs_layout_passes=False),
    )
    outt = f(code_levels.T.reshape(-1), W0.T.reshape(-1), W1.T.reshape(-1),
             W2.T.reshape(-1), W3)
    return outt.T
